# baseline (speedup 1.0000x reference)
import jax
import jax.numpy as jnp
from jax import lax
from jax.experimental import pallas as pl
from jax.experimental.pallas import tpu as pltpu
from jax.experimental.pallas import tpu_sc as plsc

BATCH = 16384
EMB_DIM = 64
N_USERS_HALF_PAIRS = 250000
_INFO = plsc.get_sparse_core_info()
_NC, _NS, _L = _INFO.num_cores, _INFO.num_subcores, _INFO.num_lanes
_NW = _NC * _NS
_BPW = BATCH // _NW
_NCHUNK = _BPW // _L
_DEPTH = 3


def _body(uid_hbm, bid_hbm, ulo_hbm, uhi_hbm, btab_hbm, out_hbm,
          uid_v, bid_v,
          ul0, ul1, ul2, uh0, uh1, uh2, bb0, bb1, bb2, out_v,
          sl0, sl1, sl2, sh0, sh1, sh2, sb0, sb1, sb2):
    wid = lax.axis_index("s") * _NC + lax.axis_index("c")
    base = wid * _BPW

    pltpu.sync_copy(uid_hbm.at[pl.ds(base, _BPW)], uid_v)
    pltpu.sync_copy(bid_hbm.at[pl.ds(base, _BPW)], bid_v)

    ulbufs, uhbufs, bbufs = (ul0, ul1, ul2), (uh0, uh1, uh2), (bb0, bb1, bb2)
    lsems, hsems, bsems = (sl0, sl1, sl2), (sh0, sh1, sh2), (sb0, sb1, sb2)
    lane = lax.iota(jnp.int32, _L)
    nhalf = N_USERS_HALF_PAIRS

    def ids(c):
        return uid_v[pl.ds(c * _L, _L)], bid_v[pl.ds(c * _L, _L)]

    def compute(c, k):
        uvec, bvec = ids(c)
        in_lo = (uvec >> 1) < nhalf
        uhalf = (uvec & 1) << 6
        bhalf = (bvec & 1) << 6

        def step(d, acc):
            col = lax.bitwise_and(d + lane, EMB_DIM - 1)
            ul = plsc.load_gather(ulbufs[k], [lane, uhalf + col])
            uh = plsc.load_gather(uhbufs[k], [lane, uhalf + col])
            u = jnp.where(in_lo, ul, uh)
            b = plsc.load_gather(bbufs[k], [lane, bhalf + col])
            return acc + u * b

        acc = lax.fori_loop(0, EMB_DIM, step, jnp.zeros((_L,), jnp.float32))
        out_v[pl.ds(c * _L, _L)] = acc

    def fire(c, k):
        uvec, bvec = ids(c)
        q = uvec >> 1
        qlo = jnp.minimum(q, nhalf - 1)
        qhi = jnp.clip(q - nhalf, 0, nhalf - 1)
        return (
            pltpu.async_copy(ulo_hbm.at[qlo], ulbufs[k], lsems[k]),
            pltpu.async_copy(uhi_hbm.at[qhi], uhbufs[k], hsems[k]),
            pltpu.async_copy(btab_hbm.at[bvec >> 1], bbufs[k], bsems[k]),
        )

    def stage(t, nfire):
        c0 = t * _DEPTH
        fired = [fire(c0 + s, s) for s in range(nfire)]
        for s in range(nfire):
            for cp in fired[s]:
                cp.wait()
            compute(c0 + s, s)
        return 0

    lax.fori_loop(0, _NCHUNK // _DEPTH, lambda t, x: stage(t, _DEPTH), 0)
    if _NCHUNK % _DEPTH:
        stage(_NCHUNK // _DEPTH, _NCHUNK % _DEPTH)

    pltpu.sync_copy(out_v, out_hbm.at[pl.ds(base, _BPW)])


@jax.jit
def _run(uid, bid, ulo, uhi, btab):
    mesh = plsc.VectorSubcoreMesh(core_axis_name="c", subcore_axis_name="s")
    row_buf = pltpu.VMEM((_L, 2 * EMB_DIM), jnp.float32)
    return pl.kernel(
        _body,
        mesh=mesh,
        compiler_params=pltpu.CompilerParams(needs_layout_passes=False),
        out_type=jax.ShapeDtypeStruct((BATCH,), jnp.float32),
        scratch_types=[
            pltpu.VMEM((_BPW,), jnp.int32),
            pltpu.VMEM((_BPW,), jnp.int32),
            row_buf, row_buf, row_buf,
            row_buf, row_buf, row_buf,
            row_buf, row_buf, row_buf,
            pltpu.VMEM((_BPW,), jnp.float32),
            pltpu.SemaphoreType.DMA, pltpu.SemaphoreType.DMA,
            pltpu.SemaphoreType.DMA, pltpu.SemaphoreType.DMA,
            pltpu.SemaphoreType.DMA, pltpu.SemaphoreType.DMA,
            pltpu.SemaphoreType.DMA, pltpu.SemaphoreType.DMA,
            pltpu.SemaphoreType.DMA,
        ],
    )(uid, bid, ulo, uhi, btab)


def kernel(user_ids, banner_ids, user_table, banner_table):
    half_rows = user_table.shape[0] // 2
    ulo = user_table[:half_rows].reshape(-1, 2 * EMB_DIM)
    uhi = user_table[half_rows:].reshape(-1, 2 * EMB_DIM)
    btab2 = banner_table.reshape(-1, 2 * EMB_DIM)
    return _run(user_ids.astype(jnp.int32), banner_ids.astype(jnp.int32),
                ulo, uhi, btab2)


# speedup vs baseline: 4.6387x; 4.6387x over previous
import jax
import jax.numpy as jnp
from jax import lax
from jax.experimental import pallas as pl
from jax.experimental.pallas import tpu as pltpu
from jax.experimental.pallas import tpu_sc as plsc

BATCH = 16384
EMB_DIM = 64
_INFO = plsc.get_sparse_core_info()
_NC, _NS, _L = _INFO.num_cores, _INFO.num_subcores, _INFO.num_lanes
_NW = _NC * _NS
_BPW = BATCH // _NW
_NCHUNK = _BPW // _L
_DEPTH = 3


def _body(uid_hbm, bid_hbm, utab_hbm, btab_hbm, out_hbm,
          uid_v, bid_v, ub0, ub1, ub2, bb0, bb1, bb2, out_v,
          us0, us1, us2, bs0, bs1, bs2):
    wid = lax.axis_index("s") * _NC + lax.axis_index("c")
    base = wid * _BPW

    pltpu.sync_copy(uid_hbm.at[pl.ds(base, _BPW)], uid_v)
    pltpu.sync_copy(bid_hbm.at[pl.ds(base, _BPW)], bid_v)

    ubufs, bbufs = (ub0, ub1, ub2), (bb0, bb1, bb2)
    usems, bsems = (us0, us1, us2), (bs0, bs1, bs2)
    lane = lax.iota(jnp.int32, _L)

    def ids(c):
        return uid_v[pl.ds(c * _L, _L)], bid_v[pl.ds(c * _L, _L)]

    def compute(c, k):
        def step(d, acc):
            col = lax.bitwise_and(d + lane, EMB_DIM - 1)
            u = plsc.load_gather(ubufs[k], [lane, col])
            b = plsc.load_gather(bbufs[k], [lane, col])
            return acc + u * b

        acc = lax.fori_loop(0, EMB_DIM, step,
                            jnp.zeros((_L,), jnp.float32), unroll=4)
        out_v[pl.ds(c * _L, _L)] = acc

    def fire(c, k):
        uvec, bvec = ids(c)
        ublk, urow = uvec >> 3, uvec & 7
        bblk, brow = bvec >> 3, bvec & 7
        copies = []
        for j in range(_L):
            copies.append(pltpu.async_copy(
                utab_hbm.at[ublk[j], pl.ds(urow[j], 1)],
                ubufs[k].at[pl.ds(j, 1)], usems[k]))
            copies.append(pltpu.async_copy(
                btab_hbm.at[bblk[j], pl.ds(brow[j], 1)],
                bbufs[k].at[pl.ds(j, 1)], bsems[k]))
        return copies

    def stage(t, nfire):
        c0 = t * _DEPTH
        fired = [fire(c0 + s, s) for s in range(nfire)]
        for s in range(nfire):
            for cp in fired[s]:
                cp.wait()
            compute(c0 + s, s)
        return 0

    lax.fori_loop(0, _NCHUNK // _DEPTH, lambda t, x: stage(t, _DEPTH), 0)
    if _NCHUNK % _DEPTH:
        stage(_NCHUNK // _DEPTH, _NCHUNK % _DEPTH)

    pltpu.sync_copy(out_v, out_hbm.at[pl.ds(base, _BPW)])


@jax.jit
def _run(uid, bid, utab3, btab3):
    mesh = plsc.VectorSubcoreMesh(core_axis_name="c", subcore_axis_name="s")
    row_buf = pltpu.VMEM((_L, EMB_DIM), jnp.float32)
    return pl.kernel(
        _body,
        mesh=mesh,
        compiler_params=pltpu.CompilerParams(needs_layout_passes=False),
        out_type=jax.ShapeDtypeStruct((BATCH,), jnp.float32),
        scratch_types=[
            pltpu.VMEM((_BPW,), jnp.int32),
            pltpu.VMEM((_BPW,), jnp.int32),
            row_buf, row_buf, row_buf,
            row_buf, row_buf, row_buf,
            pltpu.VMEM((_BPW,), jnp.float32),
            pltpu.SemaphoreType.DMA, pltpu.SemaphoreType.DMA,
            pltpu.SemaphoreType.DMA, pltpu.SemaphoreType.DMA,
            pltpu.SemaphoreType.DMA, pltpu.SemaphoreType.DMA,
        ],
    )(uid, bid, utab3, btab3)


def kernel(user_ids, banner_ids, user_table, banner_table):
    utab3 = user_table.reshape(-1, 8, EMB_DIM)
    btab3 = banner_table.reshape(-1, 8, EMB_DIM)
    return _run(user_ids.astype(jnp.int32), banner_ids.astype(jnp.int32),
                utab3, btab3)


# speedup vs baseline: 4.6794x; 1.0088x over previous
import jax
import jax.numpy as jnp
from jax import lax
from jax.experimental import pallas as pl
from jax.experimental.pallas import tpu as pltpu
from jax.experimental.pallas import tpu_sc as plsc

BATCH = 16384
EMB_DIM = 64
_INFO = plsc.get_sparse_core_info()
_NC, _NS, _L = _INFO.num_cores, _INFO.num_subcores, _INFO.num_lanes
_NW = _NC * _NS
_BPW = BATCH // _NW
_NCHUNK = _BPW // _L
_DEPTH = 4


def _body(uid_hbm, bid_hbm, utab_hbm, btab_hbm, out_hbm,
          uid_v, bid_v, ub0, ub1, ub2, ub3, bb0, bb1, bb2, bb3, out_v,
          us0, us1, us2, us3, bs0, bs1, bs2, bs3):
    wid = lax.axis_index("s") * _NC + lax.axis_index("c")
    base = wid * _BPW

    pltpu.sync_copy(uid_hbm.at[pl.ds(base, _BPW)], uid_v)
    pltpu.sync_copy(bid_hbm.at[pl.ds(base, _BPW)], bid_v)

    ubufs, bbufs = (ub0, ub1, ub2, ub3), (bb0, bb1, bb2, bb3)
    usems, bsems = (us0, us1, us2, us3), (bs0, bs1, bs2, bs3)
    lane = lax.iota(jnp.int32, _L)

    def ids(c):
        return uid_v[pl.ds(c * _L, _L)], bid_v[pl.ds(c * _L, _L)]

    def compute(c, k):
        def step(d, acc):
            col = lax.bitwise_and(d + lane, EMB_DIM - 1)
            u = plsc.load_gather(ubufs[k], [lane, col])
            b = plsc.load_gather(bbufs[k], [lane, col])
            return acc + u * b

        acc = lax.fori_loop(0, EMB_DIM, step,
                            jnp.zeros((_L,), jnp.float32), unroll=8)
        out_v[pl.ds(c * _L, _L)] = acc

    def fire(c, k):
        uvec, bvec = ids(c)
        ublk, urow = uvec >> 3, uvec & 7
        bblk, brow = bvec >> 3, bvec & 7
        copies = []
        for j in range(_L):
            copies.append(pltpu.async_copy(
                utab_hbm.at[ublk[j], pl.ds(urow[j], 1)],
                ubufs[k].at[pl.ds(j, 1)], usems[k]))
            copies.append(pltpu.async_copy(
                btab_hbm.at[bblk[j], pl.ds(brow[j], 1)],
                bbufs[k].at[pl.ds(j, 1)], bsems[k]))
        return copies

    def stage(t, nfire):
        c0 = t * _DEPTH
        fired = [fire(c0 + s, s) for s in range(nfire)]
        for s in range(nfire):
            for cp in fired[s]:
                cp.wait()
            compute(c0 + s, s)
        return 0

    lax.fori_loop(0, _NCHUNK // _DEPTH, lambda t, x: stage(t, _DEPTH), 0)
    if _NCHUNK % _DEPTH:
        stage(_NCHUNK // _DEPTH, _NCHUNK % _DEPTH)

    pltpu.sync_copy(out_v, out_hbm.at[pl.ds(base, _BPW)])


@jax.jit
def _run(uid, bid, utab3, btab3):
    mesh = plsc.VectorSubcoreMesh(core_axis_name="c", subcore_axis_name="s")
    row_buf = pltpu.VMEM((_L, EMB_DIM), jnp.float32)
    return pl.kernel(
        _body,
        mesh=mesh,
        compiler_params=pltpu.CompilerParams(needs_layout_passes=False),
        out_type=jax.ShapeDtypeStruct((BATCH,), jnp.float32),
        scratch_types=[
            pltpu.VMEM((_BPW,), jnp.int32),
            pltpu.VMEM((_BPW,), jnp.int32),
            row_buf, row_buf, row_buf, row_buf,
            row_buf, row_buf, row_buf, row_buf,
            pltpu.VMEM((_BPW,), jnp.float32),
            pltpu.SemaphoreType.DMA, pltpu.SemaphoreType.DMA,
            pltpu.SemaphoreType.DMA, pltpu.SemaphoreType.DMA,
            pltpu.SemaphoreType.DMA, pltpu.SemaphoreType.DMA,
            pltpu.SemaphoreType.DMA, pltpu.SemaphoreType.DMA,
        ],
    )(uid, bid, utab3, btab3)


def kernel(user_ids, banner_ids, user_table, banner_table):
    utab3 = user_table.reshape(-1, 8, EMB_DIM)
    btab3 = banner_table.reshape(-1, 8, EMB_DIM)
    return _run(user_ids.astype(jnp.int32), banner_ids.astype(jnp.int32),
                utab3, btab3)
